# Initial kernel scaffold; baseline (speedup 1.0000x reference)
import jax
import jax.numpy as jnp
from jax.experimental import pallas as pl


def kernel(inputs, hx, ru_weights, ru_biases, gconv_weights, gconv_biases, s1_rows, s1_cols, s1_vals, s2_rows, s2_cols, s2_vals):
    raise NotImplementedError("write your pallas kernel here")


# speedup vs baseline: 3.4694x; 3.4694x over previous
import jax
import jax.numpy as jnp
from jax.experimental import pallas as pl

_N = 2048
_B = 32
_U = 64
_I = 2
_F = _U + _I
_FP = 68
_NM = 5
_CB = 128
_NB = 256


def _diffusion_body(s1_ref, s2_ref, x_ref, o1_ref, o2_ref, o3_ref, o4_ref):
    s1 = s1_ref[...]
    s2 = s2_ref[...]
    x0 = x_ref[...].astype(jnp.float32)
    y1 = jnp.dot(s1, x_ref[...], preferred_element_type=jnp.float32)
    y1b = y1.astype(jnp.bfloat16)
    y2 = 2.0 * jnp.dot(s1, y1b, preferred_element_type=jnp.float32) - x0
    y3 = jnp.dot(s2, y1b, preferred_element_type=jnp.float32)
    y4 = 2.0 * jnp.dot(s2, y3.astype(jnp.bfloat16),
                       preferred_element_type=jnp.float32) - y1
    o1_ref[...] = y1b
    o2_ref[...] = y2.astype(jnp.bfloat16)
    o3_ref[...] = y3.astype(jnp.bfloat16)
    o4_ref[...] = y4.astype(jnp.bfloat16)


def _diffusion(s1d, s2d, x0):
    w = x0.shape[1]
    grid = (w // _CB,)
    full = pl.BlockSpec((_N, _N), lambda j: (0, 0))
    colb = pl.BlockSpec((_N, _CB), lambda j: (0, j))
    out = jax.ShapeDtypeStruct((_N, w), jnp.bfloat16)
    return pl.pallas_call(
        _diffusion_body,
        grid=grid,
        in_specs=[full, full, colb],
        out_specs=[colb, colb, colb, colb],
        out_shape=[out, out, out, out],
    )(s1d, s2d, x0)


def _ru_body(w_ref, b_ref, x0_ref, m1_ref, m2_ref, m3_ref, m4_ref, hx_ref,
             xi_ref, x0p_ref, u_ref):
    acc = b_ref[...].astype(jnp.float32)
    for k, ref in enumerate((x0_ref, m1_ref, m2_ref, m3_ref, m4_ref)):
        acc = acc + jnp.dot(ref[:, 0, 0, :], w_ref[k],
                            preferred_element_type=jnp.float32)
    val = jax.nn.sigmoid(acc)
    r = val[:, :_U]
    u = val[:, _U:]
    rhx = (r * hx_ref[0]).astype(jnp.bfloat16)
    pad = jnp.zeros((rhx.shape[0], _FP - _F), jnp.bfloat16)
    x0p_ref[:, 0, 0, :] = jnp.concatenate([xi_ref[:, 0, 0, :], rhx, pad],
                                          axis=1)
    u_ref[0] = u


def _gout_body(w_ref, b_ref, x0_ref, m1_ref, m2_ref, m3_ref, m4_ref,
               u_ref, hx_ref, o_ref):
    acc = b_ref[...].astype(jnp.float32)
    for k, ref in enumerate((x0_ref, m1_ref, m2_ref, m3_ref, m4_ref)):
        acc = acc + jnp.dot(ref[:, 0, 0, :], w_ref[k],
                            preferred_element_type=jnp.float32)
    c = jnp.tanh(acc)
    u = u_ref[0]
    o_ref[0] = u * hx_ref[0] + (1.0 - u) * c


def _split_weights(w, out_dim):
    wm = jnp.transpose(w.reshape(_F, _NM, out_dim), (1, 0, 2))
    wm = jnp.pad(wm, ((0, 0), (0, _FP - _F), (0, 0)))
    return wm.astype(jnp.bfloat16)


def kernel(inputs, hx, ru_weights, ru_biases, gconv_weights, gconv_biases,
           s1_rows, s1_cols, s1_vals, s2_rows, s2_cols, s2_vals):
    s1d = jnp.zeros((_N, _N), jnp.float32).at[s1_rows, s1_cols].add(
        s1_vals).astype(jnp.bfloat16)
    s2d = jnp.zeros((_N, _N), jnp.float32).at[s2_rows, s2_cols].add(
        s2_vals).astype(jnp.bfloat16)

    xi = inputs.reshape(_B, _N, _I).transpose(1, 0, 2)
    hx3 = hx.reshape(_B, _N, _U)
    x0 = jnp.concatenate([xi, hx3.transpose(1, 0, 2)], axis=2)
    x0 = jnp.pad(x0, ((0, 0), (0, 0), (0, _FP - _F)))
    x0b = x0.reshape(_N, _B * _FP).astype(jnp.bfloat16)
    xib = xi.astype(jnp.bfloat16).reshape(_N, _B, 1, _I)

    w_ru = _split_weights(ru_weights, 2 * _U)
    w_g = _split_weights(gconv_weights, _U)
    b_ru = ru_biases.reshape(1, 2 * _U)
    b_g = gconv_biases.reshape(1, _U)

    m1, m2, m3, m4 = _diffusion(s1d, s2d, x0b)

    grid = (_B, _N // _NB)
    wspec2 = pl.BlockSpec((_NM, _FP, 2 * _U), lambda b, n: (0, 0, 0))
    wspec1 = pl.BlockSpec((_NM, _FP, _U), lambda b, n: (0, 0, 0))
    bspec2 = pl.BlockSpec((1, 2 * _U), lambda b, n: (0, 0))
    bspec1 = pl.BlockSpec((1, _U), lambda b, n: (0, 0))
    nmat = pl.BlockSpec((_NB, 1, 1, _FP), lambda b, n: (n, b, 0, 0))
    xispec = pl.BlockSpec((_NB, 1, 1, _I), lambda b, n: (n, b, 0, 0))
    bmat = pl.BlockSpec((1, _NB, _U), lambda b, n: (b, n, 0))

    def as_nbf(m):
        return m.reshape(_N, _B, 1, _FP)

    x0p, u = pl.pallas_call(
        _ru_body,
        grid=grid,
        in_specs=[wspec2, bspec2, nmat, nmat, nmat, nmat, nmat, bmat,
                  xispec],
        out_specs=[nmat, bmat],
        out_shape=[jax.ShapeDtypeStruct((_N, _B, 1, _FP), jnp.bfloat16),
                   jax.ShapeDtypeStruct((_B, _N, _U), jnp.float32)],
    )(w_ru, b_ru, as_nbf(x0b), as_nbf(m1), as_nbf(m2), as_nbf(m3),
      as_nbf(m4), hx3, xib)

    x0p = x0p.reshape(_N, _B * _FP)

    p1, p2, p3, p4 = _diffusion(s1d, s2d, x0p)

    out = pl.pallas_call(
        _gout_body,
        grid=grid,
        in_specs=[wspec1, bspec1, nmat, nmat, nmat, nmat, nmat, bmat, bmat],
        out_specs=bmat,
        out_shape=jax.ShapeDtypeStruct((_B, _N, _U), jnp.float32),
    )(w_g, b_g, as_nbf(x0p), as_nbf(p1), as_nbf(p2), as_nbf(p3), as_nbf(p4),
      u, hx3)

    return out.reshape(_B, _N * _U)


# speedup vs baseline: 7.9105x; 2.2801x over previous
import jax
import jax.numpy as jnp
from jax.experimental import pallas as pl
from jax.experimental.pallas import tpu as pltpu

_PARAMS = pltpu.CompilerParams(vmem_limit_bytes=100 * 1024 * 1024)

_N = 2048
_B = 32
_U = 64
_I = 2
_NM = 5
_GB = 2


def _chain(s1, s2, x):
    x32 = x.astype(jnp.float32)
    y1 = jnp.dot(s1, x, preferred_element_type=jnp.float32)
    y1b = y1.astype(jnp.bfloat16)
    y2 = 2.0 * jnp.dot(s1, y1b, preferred_element_type=jnp.float32) - x32
    y3 = jnp.dot(s2, y1b, preferred_element_type=jnp.float32)
    y3b = y3.astype(jnp.bfloat16)
    y4 = 2.0 * jnp.dot(s2, y3b, preferred_element_type=jnp.float32) - y1
    return y1b, y2.astype(jnp.bfloat16), y3b, y4.astype(jnp.bfloat16)


def _xi_body(s1_ref, s2_ref, xi_ref, o1_ref, o2_ref, o3_ref, o4_ref):
    y1, y2, y3, y4 = _chain(s1_ref[...], s2_ref[...], xi_ref[...])
    o1_ref[...] = y1
    o2_ref[...] = y2
    o3_ref[...] = y3
    o4_ref[...] = y4


def _ru_body(s1_ref, s2_ref, xic_ref, wi_ref, ws_ref, b_ref, hx_ref,
             xsp_ref, u_ref):
    s1 = s1_ref[...]
    s2 = s2_ref[...]
    hxb = [hx_ref[jb] for jb in range(_GB)]
    hxb16 = [h.astype(jnp.bfloat16) for h in hxb]
    xs = jnp.concatenate(hxb16, axis=1)
    ys = _chain(s1, s2, xs)
    wi = wi_ref[...]
    bias = b_ref[...].astype(jnp.float32)
    rhx = []
    for jb in range(_GB):
        acc = bias + jnp.dot(xic_ref[jb], wi,
                             preferred_element_type=jnp.float32)
        acc = acc + jnp.dot(hxb16[jb], ws_ref[0],
                            preferred_element_type=jnp.float32)
        for m in range(1, _NM):
            acc = acc + jnp.dot(ys[m - 1][:, jb * _U:(jb + 1) * _U],
                                ws_ref[m], preferred_element_type=jnp.float32)
        val = jax.nn.sigmoid(acc)
        r = val[:, :_U]
        u_ref[jb] = val[:, _U:]
        rhx.append((r * hxb[jb]).astype(jnp.bfloat16))
    xsp_ref[...] = jnp.concatenate(rhx, axis=1)


def _gout_body(s1_ref, s2_ref, xic_ref, wi_ref, ws_ref, b_ref, xsp_ref,
               u_ref, hx_ref, o_ref):
    s1 = s1_ref[...]
    s2 = s2_ref[...]
    xs = xsp_ref[...]
    ys = _chain(s1, s2, xs)
    wi = wi_ref[...]
    bias = b_ref[...].astype(jnp.float32)
    for jb in range(_GB):
        acc = bias + jnp.dot(xic_ref[jb], wi,
                             preferred_element_type=jnp.float32)
        acc = acc + jnp.dot(xs[:, jb * _U:(jb + 1) * _U], ws_ref[0],
                            preferred_element_type=jnp.float32)
        for m in range(1, _NM):
            acc = acc + jnp.dot(ys[m - 1][:, jb * _U:(jb + 1) * _U],
                                ws_ref[m], preferred_element_type=jnp.float32)
        c = jnp.tanh(acc)
        u = u_ref[jb]
        o_ref[jb] = u * hx_ref[jb] + (1.0 - u) * c


def _split_weights(w, out_dim):
    w3 = w.reshape(_I + _U, _NM, out_dim)
    wi = jnp.transpose(w3[:_I], (1, 0, 2)).reshape(_NM * _I, out_dim)
    ws = jnp.transpose(w3[_I:], (1, 0, 2))
    return wi.astype(jnp.bfloat16), ws.astype(jnp.bfloat16)


def kernel(inputs, hx, ru_weights, ru_biases, gconv_weights, gconv_biases,
           s1_rows, s1_cols, s1_vals, s2_rows, s2_cols, s2_vals):
    s1d = jnp.zeros((_N, _N), jnp.float32).at[s1_rows, s1_cols].add(
        s1_vals).astype(jnp.bfloat16)
    s2d = jnp.zeros((_N, _N), jnp.float32).at[s2_rows, s2_cols].add(
        s2_vals).astype(jnp.bfloat16)

    xi = inputs.reshape(_B, _N, _I).transpose(1, 0, 2).reshape(_N, _B * _I)
    xib = xi.astype(jnp.bfloat16)
    hx3 = hx.reshape(_B, _N, _U)

    wi_ru, ws_ru = _split_weights(ru_weights, 2 * _U)
    wi_g, ws_g = _split_weights(gconv_weights, _U)
    b_ru = ru_biases.reshape(1, 2 * _U)
    b_g = gconv_biases.reshape(1, _U)

    full = pl.BlockSpec((_N, _N), lambda j: (0, 0))
    xifull = pl.BlockSpec((_N, _B * _I), lambda j: (0, 0))
    xm1, xm2, xm3, xm4 = pl.pallas_call(
        _xi_body,
        grid=(1,),
        in_specs=[full, full, xifull],
        out_specs=[xifull] * 4,
        out_shape=[jax.ShapeDtypeStruct((_N, _B * _I), jnp.bfloat16)] * 4,
        compiler_params=_PARAMS,
    )(s1d, s2d, xib)

    xic = jnp.stack(
        [a.reshape(_N, _B, _I) for a in (xib, xm1, xm2, xm3, xm4)], axis=2
    ).reshape(_N, _B, _NM * _I).transpose(1, 0, 2)

    grid = (_B // _GB,)
    xicspec = pl.BlockSpec((_GB, _N, _NM * _I), lambda j: (j, 0, 0))
    wispec2 = pl.BlockSpec((_NM * _I, 2 * _U), lambda j: (0, 0))
    wsspec2 = pl.BlockSpec((_NM, _U, 2 * _U), lambda j: (0, 0, 0))
    wispec1 = pl.BlockSpec((_NM * _I, _U), lambda j: (0, 0))
    wsspec1 = pl.BlockSpec((_NM, _U, _U), lambda j: (0, 0, 0))
    bspec2 = pl.BlockSpec((1, 2 * _U), lambda j: (0, 0))
    bspec1 = pl.BlockSpec((1, _U), lambda j: (0, 0))
    bblk = pl.BlockSpec((_GB, _N, _U), lambda j: (j, 0, 0))
    xsblk = pl.BlockSpec((_N, _GB * _U), lambda j: (0, j))

    xsp, u = pl.pallas_call(
        _ru_body,
        grid=grid,
        in_specs=[full, full, xicspec, wispec2, wsspec2, bspec2, bblk],
        out_specs=[xsblk, bblk],
        out_shape=[jax.ShapeDtypeStruct((_N, _B * _U), jnp.bfloat16),
                   jax.ShapeDtypeStruct((_B, _N, _U), jnp.float32)],
        compiler_params=_PARAMS,
    )(s1d, s2d, xic, wi_ru, ws_ru, b_ru, hx3)

    out = pl.pallas_call(
        _gout_body,
        grid=grid,
        in_specs=[full, full, xicspec, wispec1, wsspec1, bspec1, xsblk,
                  bblk, bblk],
        out_specs=bblk,
        out_shape=jax.ShapeDtypeStruct((_B, _N, _U), jnp.float32),
        compiler_params=_PARAMS,
    )(s1d, s2d, xic, wi_g, ws_g, b_g, xsp, u, hx3)

    return out.reshape(_B, _N * _U)


# speedup vs baseline: 12.6623x; 1.6007x over previous
import functools

import jax
import jax.numpy as jnp
from jax import lax
from jax.experimental import pallas as pl
from jax.experimental.pallas import tpu as pltpu
from jax.experimental.pallas import tpu_sc as plsc

_PARAMS = pltpu.CompilerParams(vmem_limit_bytes=100 * 1024 * 1024)

_N = 2048
_B = 32
_U = 64
_I = 2
_NM = 5
_GB = 2


_MAXN = 4096
_RC = 32
_NW = 32


def _densify_body(r1, c1, v1, r2, c2, v2, of1, of2, o1, o2,
                  rowbuf, rstg, cstg, vstg, ostg):
    wid = lax.axis_index("s") * 2 + lax.axis_index("c")
    zeros = jnp.zeros((16,), jnp.float32)
    for (r_h, c_h, v_h, o_h, of_h) in ((r1, c1, v1, o1, of1),
                                       (r2, c2, v2, o2, of2)):
        pltpu.sync_copy(of_h, ostg)
        for cix in range(2):
            k = wid + _NW * cix
            base = k * _RC
            kvec = jnp.zeros((16,), jnp.int32) + k
            s0 = pl.multiple_of(
                jnp.max(plsc.load_gather(ostg, [kvec])) & ~15, 16)
            pltpu.sync_copy(r_h.at[pl.ds(s0, _MAXN)], rstg)
            pltpu.sync_copy(c_h.at[pl.ds(s0, _MAXN)], cstg)
            pltpu.sync_copy(v_h.at[pl.ds(s0, _MAXN)], vstg)

            def zero_body(t, _):
                rowbuf[pl.ds(t * 16, 16)] = zeros
                return 0

            lax.fori_loop(0, _RC * 2048 // 16, zero_body, 0)

            basev = jnp.zeros((16,), jnp.int32) + base

            def scat_body(t, _):
                rv = rstg[pl.ds(t * 16, 16)]
                cv = cstg[pl.ds(t * 16, 16)]
                vv = vstg[pl.ds(t * 16, 16)]
                msk = (rv >= base) & (rv < base + _RC)
                idx = (rv - basev) * 2048 + cv
                plsc.store_scatter(rowbuf, [idx], vv, mask=msk)
                return 0

            lax.fori_loop(0, _MAXN // 16, scat_body, 0)
            pltpu.sync_copy(rowbuf, o_h.at[pl.ds(base * 2048, _RC * 2048)])


def _densify(s1_rows, s1_cols, s1_vals, s2_rows, s2_cols, s2_vals):
    bound = jnp.arange(0, _N + 1, _RC, dtype=jnp.int32)
    pads = (jnp.full((_MAXN + 16,), jnp.int32(1 << 24)),
            jnp.zeros((_MAXN + 16,), jnp.float32))

    def prep(rows, cols, vals):
        offs = jnp.searchsorted(rows, bound).astype(jnp.int32)
        offs = jnp.pad(offs, (0, 7))
        rp = jnp.concatenate([rows, pads[0]])
        cp = jnp.concatenate([cols, jnp.zeros_like(pads[0])])
        vp = jnp.concatenate([vals, pads[1]])
        return rp, cp, vp, offs

    rp1, cp1, vp1, of1 = prep(s1_rows, s1_cols, s1_vals)
    rp2, cp2, vp2, of2 = prep(s2_rows, s2_cols, s2_vals)
    mesh = plsc.VectorSubcoreMesh(core_axis_name="c", subcore_axis_name="s")
    kfn = pl.kernel(
        _densify_body,
        out_type=[jax.ShapeDtypeStruct((_N * _N,), jnp.float32)] * 2,
        mesh=mesh,
        scratch_types=[
            pltpu.VMEM((_RC * 2048,), jnp.float32),
            pltpu.VMEM((_MAXN,), jnp.int32),
            pltpu.VMEM((_MAXN,), jnp.int32),
            pltpu.VMEM((_MAXN,), jnp.float32),
            pltpu.VMEM((72,), jnp.int32),
        ],
        compiler_params=pltpu.CompilerParams(needs_layout_passes=False),
    )
    o1, o2 = kfn(rp1, cp1, vp1, rp2, cp2, vp2, of1, of2)
    return o1.reshape(_N, _N), o2.reshape(_N, _N)


def _chain(s1, s2, x):
    x32 = x.astype(jnp.float32)
    y1 = jnp.dot(s1, x, preferred_element_type=jnp.float32)
    y1b = y1.astype(jnp.bfloat16)
    y2 = 2.0 * jnp.dot(s1, y1b, preferred_element_type=jnp.float32) - x32
    y3 = jnp.dot(s2, y1b, preferred_element_type=jnp.float32)
    y3b = y3.astype(jnp.bfloat16)
    y4 = 2.0 * jnp.dot(s2, y3b, preferred_element_type=jnp.float32) - y1
    return y1b, y2.astype(jnp.bfloat16), y3b, y4.astype(jnp.bfloat16)


def _xi_body(s1_ref, s2_ref, xi_ref, o1_ref, o2_ref, o3_ref, o4_ref):
    y1, y2, y3, y4 = _chain(s1_ref[...], s2_ref[...], xi_ref[...])
    o1_ref[...] = y1
    o2_ref[...] = y2
    o3_ref[...] = y3
    o4_ref[...] = y4


def _ru_body(s1_ref, s2_ref, xic_ref, wi_ref, ws_ref, b_ref, hx_ref,
             xsp_ref, u_ref):
    s1 = s1_ref[...]
    s2 = s2_ref[...]
    hxb = [hx_ref[jb] for jb in range(_GB)]
    hxb16 = [h.astype(jnp.bfloat16) for h in hxb]
    xs = jnp.concatenate(hxb16, axis=1)
    ys = _chain(s1, s2, xs)
    wi = wi_ref[...]
    bias = b_ref[...].astype(jnp.float32)
    rhx = []
    for jb in range(_GB):
        acc = bias + jnp.dot(xic_ref[jb], wi,
                             preferred_element_type=jnp.float32)
        acc = acc + jnp.dot(hxb16[jb], ws_ref[0],
                            preferred_element_type=jnp.float32)
        for m in range(1, _NM):
            acc = acc + jnp.dot(ys[m - 1][:, jb * _U:(jb + 1) * _U],
                                ws_ref[m], preferred_element_type=jnp.float32)
        val = jax.nn.sigmoid(acc)
        r = val[:, :_U]
        u_ref[jb] = val[:, _U:]
        rhx.append((r * hxb[jb]).astype(jnp.bfloat16))
    xsp_ref[...] = jnp.concatenate(rhx, axis=1)


def _gout_body(s1_ref, s2_ref, xic_ref, wi_ref, ws_ref, b_ref, xsp_ref,
               u_ref, hx_ref, o_ref):
    s1 = s1_ref[...]
    s2 = s2_ref[...]
    xs = xsp_ref[...]
    ys = _chain(s1, s2, xs)
    wi = wi_ref[...]
    bias = b_ref[...].astype(jnp.float32)
    for jb in range(_GB):
        acc = bias + jnp.dot(xic_ref[jb], wi,
                             preferred_element_type=jnp.float32)
        acc = acc + jnp.dot(xs[:, jb * _U:(jb + 1) * _U], ws_ref[0],
                            preferred_element_type=jnp.float32)
        for m in range(1, _NM):
            acc = acc + jnp.dot(ys[m - 1][:, jb * _U:(jb + 1) * _U],
                                ws_ref[m], preferred_element_type=jnp.float32)
        c = jnp.tanh(acc)
        u = u_ref[jb]
        o_ref[jb] = u * hx_ref[jb] + (1.0 - u) * c


def _split_weights(w, out_dim):
    w3 = w.reshape(_I + _U, _NM, out_dim)
    wi = jnp.transpose(w3[:_I], (1, 0, 2)).reshape(_NM * _I, out_dim)
    ws = jnp.transpose(w3[_I:], (1, 0, 2))
    return wi.astype(jnp.bfloat16), ws.astype(jnp.bfloat16)


def kernel(inputs, hx, ru_weights, ru_biases, gconv_weights, gconv_biases,
           s1_rows, s1_cols, s1_vals, s2_rows, s2_cols, s2_vals):
    s1f, s2f = _densify(s1_rows, s1_cols, s1_vals,
                        s2_rows, s2_cols, s2_vals)
    s1d = s1f.astype(jnp.bfloat16)
    s2d = s2f.astype(jnp.bfloat16)

    xi = inputs.reshape(_B, _N, _I).transpose(1, 0, 2).reshape(_N, _B * _I)
    xib = xi.astype(jnp.bfloat16)
    hx3 = hx.reshape(_B, _N, _U)

    wi_ru, ws_ru = _split_weights(ru_weights, 2 * _U)
    wi_g, ws_g = _split_weights(gconv_weights, _U)
    b_ru = ru_biases.reshape(1, 2 * _U)
    b_g = gconv_biases.reshape(1, _U)

    full = pl.BlockSpec((_N, _N), lambda j: (0, 0))
    xifull = pl.BlockSpec((_N, _B * _I), lambda j: (0, 0))
    xm1, xm2, xm3, xm4 = pl.pallas_call(
        _xi_body,
        grid=(1,),
        in_specs=[full, full, xifull],
        out_specs=[xifull] * 4,
        out_shape=[jax.ShapeDtypeStruct((_N, _B * _I), jnp.bfloat16)] * 4,
        compiler_params=_PARAMS,
    )(s1d, s2d, xib)

    xic = jnp.stack(
        [a.reshape(_N, _B, _I) for a in (xib, xm1, xm2, xm3, xm4)], axis=2
    ).reshape(_N, _B, _NM * _I).transpose(1, 0, 2)

    grid = (_B // _GB,)
    xicspec = pl.BlockSpec((_GB, _N, _NM * _I), lambda j: (j, 0, 0))
    wispec2 = pl.BlockSpec((_NM * _I, 2 * _U), lambda j: (0, 0))
    wsspec2 = pl.BlockSpec((_NM, _U, 2 * _U), lambda j: (0, 0, 0))
    wispec1 = pl.BlockSpec((_NM * _I, _U), lambda j: (0, 0))
    wsspec1 = pl.BlockSpec((_NM, _U, _U), lambda j: (0, 0, 0))
    bspec2 = pl.BlockSpec((1, 2 * _U), lambda j: (0, 0))
    bspec1 = pl.BlockSpec((1, _U), lambda j: (0, 0))
    bblk = pl.BlockSpec((_GB, _N, _U), lambda j: (j, 0, 0))
    xsblk = pl.BlockSpec((_N, _GB * _U), lambda j: (0, j))

    xsp, u = pl.pallas_call(
        _ru_body,
        grid=grid,
        in_specs=[full, full, xicspec, wispec2, wsspec2, bspec2, bblk],
        out_specs=[xsblk, bblk],
        out_shape=[jax.ShapeDtypeStruct((_N, _B * _U), jnp.bfloat16),
                   jax.ShapeDtypeStruct((_B, _N, _U), jnp.float32)],
        compiler_params=_PARAMS,
    )(s1d, s2d, xic, wi_ru, ws_ru, b_ru, hx3)

    out = pl.pallas_call(
        _gout_body,
        grid=grid,
        in_specs=[full, full, xicspec, wispec1, wsspec1, bspec1, xsblk,
                  bblk, bblk],
        out_specs=bblk,
        out_shape=jax.ShapeDtypeStruct((_B, _N, _U), jnp.float32),
        compiler_params=_PARAMS,
    )(s1d, s2d, xic, wi_g, ws_g, b_g, xsp, u, hx3)

    return out.reshape(_B, _N * _U)


# speedup vs baseline: 18.2407x; 1.4405x over previous
import functools

import jax
import jax.numpy as jnp
from jax import lax
from jax.experimental import pallas as pl
from jax.experimental.pallas import tpu as pltpu
from jax.experimental.pallas import tpu_sc as plsc

_PARAMS = pltpu.CompilerParams(vmem_limit_bytes=100 * 1024 * 1024)

_N = 2048
_B = 32
_U = 64
_I = 2
_NM = 5
_GB = 4


_MAXN = 4096
_RC = 32
_NW = 32


def _densify_body(r1, c1, v1, r2, c2, v2, of1, of2, o1, o2,
                  rowbuf, rstg, cstg, vstg, ostg):
    wid = lax.axis_index("s") * 2 + lax.axis_index("c")
    zeros = jnp.zeros((16,), jnp.float32)
    for (r_h, c_h, v_h, o_h, of_h) in ((r1, c1, v1, o1, of1),
                                       (r2, c2, v2, o2, of2)):
        pltpu.sync_copy(of_h, ostg)
        for cix in range(2):
            k = wid + _NW * cix
            base = k * _RC
            kvec = jnp.zeros((16,), jnp.int32) + k
            s0 = pl.multiple_of(
                jnp.max(plsc.load_gather(ostg, [kvec])) & ~15, 16)
            pltpu.sync_copy(r_h.at[pl.ds(s0, _MAXN)], rstg)
            pltpu.sync_copy(c_h.at[pl.ds(s0, _MAXN)], cstg)
            pltpu.sync_copy(v_h.at[pl.ds(s0, _MAXN)], vstg)

            def zero_body(t, _):
                rowbuf[pl.ds(t * 16, 16)] = zeros
                return 0

            lax.fori_loop(0, _RC * 2048 // 16, zero_body, 0)

            basev = jnp.zeros((16,), jnp.int32) + base

            def scat_body(t, _):
                rv = rstg[pl.ds(t * 16, 16)]
                cv = cstg[pl.ds(t * 16, 16)]
                vv = vstg[pl.ds(t * 16, 16)]
                msk = (rv >= base) & (rv < base + _RC)
                idx = (rv - basev) * 2048 + cv
                plsc.store_scatter(rowbuf, [idx], vv, mask=msk)
                return 0

            lax.fori_loop(0, _MAXN // 16, scat_body, 0)
            pltpu.sync_copy(rowbuf, o_h.at[pl.ds(base * 2048, _RC * 2048)])


def _densify(s1_rows, s1_cols, s1_vals, s2_rows, s2_cols, s2_vals):
    bound = jnp.arange(0, _N + 1, _RC, dtype=jnp.int32)
    nnz1 = s1_rows.shape[0]
    nnz2 = s2_rows.shape[0]

    def prep(rows, nnz):
        offs = jnp.sum(rows[None, :] < bound[:, None], axis=1,
                       dtype=jnp.int32)
        offs = jnp.minimum(offs, nnz - _MAXN)
        return jnp.pad(offs, (0, 7))

    of1 = prep(s1_rows, nnz1)
    of2 = prep(s2_rows, nnz2)
    rp1, cp1, vp1 = s1_rows, s1_cols, s1_vals
    rp2, cp2, vp2 = s2_rows, s2_cols, s2_vals
    mesh = plsc.VectorSubcoreMesh(core_axis_name="c", subcore_axis_name="s")
    kfn = pl.kernel(
        _densify_body,
        out_type=[jax.ShapeDtypeStruct((_N * _N,), jnp.float32)] * 2,
        mesh=mesh,
        scratch_types=[
            pltpu.VMEM((_RC * 2048,), jnp.float32),
            pltpu.VMEM((_MAXN,), jnp.int32),
            pltpu.VMEM((_MAXN,), jnp.int32),
            pltpu.VMEM((_MAXN,), jnp.float32),
            pltpu.VMEM((72,), jnp.int32),
        ],
        compiler_params=pltpu.CompilerParams(needs_layout_passes=False),
    )
    o1, o2 = kfn(rp1, cp1, vp1, rp2, cp2, vp2, of1, of2)
    return o1.reshape(_N, _N), o2.reshape(_N, _N)


def _chain(s1, s2, x):
    y1 = jnp.dot(s1, x, preferred_element_type=jnp.float32)
    y1b = y1.astype(jnp.bfloat16)
    y2 = 2.0 * jnp.dot(s1, y1b, preferred_element_type=jnp.float32) \
        - x.astype(jnp.float32)
    y3 = jnp.dot(s2, y1b, preferred_element_type=jnp.float32)
    y3b = y3.astype(jnp.bfloat16)
    y4 = 2.0 * jnp.dot(s2, y3b, preferred_element_type=jnp.float32) \
        - y1b.astype(jnp.float32)
    return y1b, y2.astype(jnp.bfloat16), y3b, y4.astype(jnp.bfloat16)


def _xi_body(s1_ref, s2_ref, xi_ref, o1_ref, o2_ref, o3_ref, o4_ref):
    y1, y2, y3, y4 = _chain(s1_ref[...], s2_ref[...], xi_ref[...])
    o1_ref[...] = y1
    o2_ref[...] = y2
    o3_ref[...] = y3
    o4_ref[...] = y4


def _ru_body(s1_ref, s2_ref, xic_ref, wi_ref, ws_ref, b_ref, hx_ref,
             xsp_ref, u_ref):
    s1 = s1_ref[...]
    s2 = s2_ref[...]
    hxb = [hx_ref[jb] for jb in range(_GB)]
    hxb16 = [h.astype(jnp.bfloat16) for h in hxb]
    xs = jnp.concatenate(hxb16, axis=1)
    ys = _chain(s1, s2, xs)
    wi = wi_ref[...]
    bias = b_ref[...].astype(jnp.float32)
    rhx = []
    for jb in range(_GB):
        acc = bias + jnp.dot(xic_ref[jb], wi,
                             preferred_element_type=jnp.float32)
        acc = acc + jnp.dot(hxb16[jb], ws_ref[0],
                            preferred_element_type=jnp.float32)
        for m in range(1, _NM):
            acc = acc + jnp.dot(ys[m - 1][:, jb * _U:(jb + 1) * _U],
                                ws_ref[m], preferred_element_type=jnp.float32)
        val = jax.nn.sigmoid(acc)
        r = val[:, :_U]
        u_ref[jb] = val[:, _U:]
        rhx.append((r * hxb[jb]).astype(jnp.bfloat16))
    xsp_ref[...] = jnp.concatenate(rhx, axis=1)


def _gout_body(s1_ref, s2_ref, xic_ref, wi_ref, ws_ref, b_ref, xsp_ref,
               u_ref, hx_ref, o_ref):
    s1 = s1_ref[...]
    s2 = s2_ref[...]
    xs = xsp_ref[...]
    ys = _chain(s1, s2, xs)
    wi = wi_ref[...]
    bias = b_ref[...].astype(jnp.float32)
    for jb in range(_GB):
        acc = bias + jnp.dot(xic_ref[jb], wi,
                             preferred_element_type=jnp.float32)
        acc = acc + jnp.dot(xs[:, jb * _U:(jb + 1) * _U], ws_ref[0],
                            preferred_element_type=jnp.float32)
        for m in range(1, _NM):
            acc = acc + jnp.dot(ys[m - 1][:, jb * _U:(jb + 1) * _U],
                                ws_ref[m], preferred_element_type=jnp.float32)
        c = jnp.tanh(acc)
        u = u_ref[jb]
        o_ref[jb] = u * hx_ref[jb] + (1.0 - u) * c


def _split_weights(w, out_dim):
    w3 = w.reshape(_I + _U, _NM, out_dim)
    wi = jnp.transpose(w3[:_I], (1, 0, 2)).reshape(_NM * _I, out_dim)
    ws = jnp.transpose(w3[_I:], (1, 0, 2))
    return wi.astype(jnp.bfloat16), ws.astype(jnp.bfloat16)


def kernel(inputs, hx, ru_weights, ru_biases, gconv_weights, gconv_biases,
           s1_rows, s1_cols, s1_vals, s2_rows, s2_cols, s2_vals):
    s1f, s2f = _densify(s1_rows, s1_cols, s1_vals,
                        s2_rows, s2_cols, s2_vals)
    s1d = s1f.astype(jnp.bfloat16)
    s2d = s2f.astype(jnp.bfloat16)

    xi = inputs.reshape(_B, _N, _I).transpose(1, 0, 2).reshape(_N, _B * _I)
    xib = xi.astype(jnp.bfloat16)
    hx3 = hx.reshape(_B, _N, _U)

    wi_ru, ws_ru = _split_weights(ru_weights, 2 * _U)
    wi_g, ws_g = _split_weights(gconv_weights, _U)
    b_ru = ru_biases.reshape(1, 2 * _U)
    b_g = gconv_biases.reshape(1, _U)

    full = pl.BlockSpec((_N, _N), lambda j: (0, 0))
    xifull = pl.BlockSpec((_N, _B * _I), lambda j: (0, 0))
    xm1, xm2, xm3, xm4 = pl.pallas_call(
        _xi_body,
        grid=(1,),
        in_specs=[full, full, xifull],
        out_specs=[xifull] * 4,
        out_shape=[jax.ShapeDtypeStruct((_N, _B * _I), jnp.bfloat16)] * 4,
        compiler_params=_PARAMS,
    )(s1d, s2d, xib)

    xic = jnp.stack(
        [a.reshape(_N, _B, _I) for a in (xib, xm1, xm2, xm3, xm4)], axis=2
    ).reshape(_N, _B, _NM * _I).transpose(1, 0, 2)

    grid = (_B // _GB,)
    xicspec = pl.BlockSpec((_GB, _N, _NM * _I), lambda j: (j, 0, 0))
    wispec2 = pl.BlockSpec((_NM * _I, 2 * _U), lambda j: (0, 0))
    wsspec2 = pl.BlockSpec((_NM, _U, 2 * _U), lambda j: (0, 0, 0))
    wispec1 = pl.BlockSpec((_NM * _I, _U), lambda j: (0, 0))
    wsspec1 = pl.BlockSpec((_NM, _U, _U), lambda j: (0, 0, 0))
    bspec2 = pl.BlockSpec((1, 2 * _U), lambda j: (0, 0))
    bspec1 = pl.BlockSpec((1, _U), lambda j: (0, 0))
    bblk = pl.BlockSpec((_GB, _N, _U), lambda j: (j, 0, 0))
    xsblk = pl.BlockSpec((_N, _GB * _U), lambda j: (0, j))

    xsp, u = pl.pallas_call(
        _ru_body,
        grid=grid,
        in_specs=[full, full, xicspec, wispec2, wsspec2, bspec2, bblk],
        out_specs=[xsblk, bblk],
        out_shape=[jax.ShapeDtypeStruct((_N, _B * _U), jnp.bfloat16),
                   jax.ShapeDtypeStruct((_B, _N, _U), jnp.float32)],
        compiler_params=_PARAMS,
    )(s1d, s2d, xic, wi_ru, ws_ru, b_ru, hx3)

    out = pl.pallas_call(
        _gout_body,
        grid=grid,
        in_specs=[full, full, xicspec, wispec1, wsspec1, bspec1, xsblk,
                  bblk, bblk],
        out_specs=bblk,
        out_shape=jax.ShapeDtypeStruct((_B, _N, _U), jnp.float32),
        compiler_params=_PARAMS,
    )(s1d, s2d, xic, wi_g, ws_g, b_g, xsp, u, hx3)

    return out.reshape(_B, _N * _U)
